# Initial kernel scaffold; baseline (speedup 1.0000x reference)
import jax
import jax.numpy as jnp
from jax.experimental import pallas as pl


def kernel(user_emb, item_emb, edge_index, edge_norm):
    raise NotImplementedError("write your pallas kernel here")


# speedup vs baseline: 4.3160x; 4.3160x over previous
import functools

import jax
import jax.numpy as jnp
from jax import lax
from jax.experimental import pallas as pl
from jax.experimental.pallas import tpu as pltpu
from jax.experimental.pallas import tpu_sc as plsc

N_USERS = 10000
N_ITEMS = 10000
N_EDGES = 320000
D = 128

NC = 2
NS = 16
L = 16

BLK = 128
NBLK = N_EDGES // BLK
ROWS_PER_SUB = 624
ROWS_TAIL = N_USERS - NS * ROWS_PER_SUB


def _body(tab, gidx, sidx, norm, out_u, out_i, acc, gi_v, si_v, nrm_v,
          rows_v, sem):
    c = lax.axis_index("c")
    s = lax.axis_index("s")

    def zero_row(r, _):
        for k in range(D // L):
            rows_v[r, pl.ds(k * L, L)] = jnp.zeros((L,), jnp.float32)
        return 0
    lax.fori_loop(0, BLK, zero_row, 0)
    base_row = s * ROWS_PER_SUB
    for j in range(4):
        pltpu.sync_copy(rows_v, acc.at[pl.ds(base_row + j * BLK, BLK)])
    pltpu.sync_copy(rows_v.at[pl.ds(0, ROWS_PER_SUB - 4 * BLK)],
                    acc.at[pl.ds(base_row + 4 * BLK, ROWS_PER_SUB - 4 * BLK)])

    @pl.when(s == 0)
    def _():
        pltpu.sync_copy(rows_v.at[pl.ds(0, ROWS_TAIL)],
                        acc.at[pl.ds(NS * ROWS_PER_SUB, ROWS_TAIL)])
    plsc.subcore_barrier()

    nblk_mine = jnp.where(s < (NBLK % NS), NBLK // NS + 1, NBLK // NS)

    def block(b, _):
        base = (s + b * NS) * BLK
        pltpu.sync_copy(gidx.at[c, pl.ds(base, BLK)], gi_v)
        pltpu.sync_copy(sidx.at[c, pl.ds(base, BLK)], si_v)
        pltpu.sync_copy(norm.at[pl.ds(base, BLK)], nrm_v)
        pltpu.async_copy(tab.at[gi_v], rows_v, sem).wait()

        def scale_row(r, _):
            sc = plsc.load_gather(nrm_v, [jnp.full((L,), r, jnp.int32)])
            for k in range(D // L):
                rows_v[r, pl.ds(k * L, L)] = rows_v[r, pl.ds(k * L, L)] * sc
            return 0
        lax.fori_loop(0, BLK, scale_row, 0)

        pltpu.sync_copy(rows_v, acc.at[si_v], add=True)
        return 0
    lax.fori_loop(0, nblk_mine, block, 0)

    plsc.subcore_barrier()

    @pl.when(c == 0)
    def _():
        pltpu.sync_copy(acc.at[pl.ds(base_row, ROWS_PER_SUB)],
                        out_i.at[pl.ds(base_row, ROWS_PER_SUB)])

        @pl.when(s == 0)
        def _():
            pltpu.sync_copy(acc.at[pl.ds(NS * ROWS_PER_SUB, ROWS_TAIL)],
                            out_i.at[pl.ds(NS * ROWS_PER_SUB, ROWS_TAIL)])

    @pl.when(c == 1)
    def _():
        pltpu.sync_copy(acc.at[pl.ds(base_row, ROWS_PER_SUB)],
                        out_u.at[pl.ds(base_row, ROWS_PER_SUB)])

        @pl.when(s == 0)
        def _():
            pltpu.sync_copy(acc.at[pl.ds(NS * ROWS_PER_SUB, ROWS_TAIL)],
                            out_u.at[pl.ds(NS * ROWS_PER_SUB, ROWS_TAIL)])


@jax.jit
def kernel(user_emb, item_emb, edge_index, edge_norm):
    u = edge_index[0].astype(jnp.int32)
    i = edge_index[1].astype(jnp.int32)
    tab = jnp.concatenate([user_emb, item_emb], axis=0)
    gidx = jnp.stack([u, i + N_USERS], axis=0)
    sidx = jnp.stack([i, u], axis=0)

    mesh = plsc.VectorSubcoreMesh(core_axis_name="c", subcore_axis_name="s",
                                  num_cores=NC, num_subcores=NS)
    run = pl.kernel(
        _body,
        out_type=(jax.ShapeDtypeStruct((N_USERS, D), jnp.float32),
                  jax.ShapeDtypeStruct((N_ITEMS, D), jnp.float32)),
        mesh=mesh,
        compiler_params=pltpu.CompilerParams(needs_layout_passes=False),
        scratch_types=[
            pltpu.VMEM_SHARED((N_USERS, D), jnp.float32),
            pltpu.VMEM((BLK,), jnp.int32),
            pltpu.VMEM((BLK,), jnp.int32),
            pltpu.VMEM((BLK,), jnp.float32),
            pltpu.VMEM((BLK, D), jnp.float32),
            pltpu.SemaphoreType.DMA,
        ],
    )
    agg_users, agg_items = run(tab, gidx, sidx, edge_norm)
    return (agg_users, agg_items)
